# Initial kernel scaffold; baseline (speedup 1.0000x reference)
import jax
import jax.numpy as jnp
from jax.experimental import pallas as pl


def kernel(x_0, edge_index, W0_l0, b0_l0, W1_l0, b1_l0, W0_l1, b0_l1, W1_l1, b1_l1, lin_W, lin_b):
    raise NotImplementedError("write your pallas kernel here")


# speedup vs baseline: 1.0002x; 1.0002x over previous
import jax
import jax.numpy as jnp
from jax.experimental import pallas as pl

N = 10000
EH = 10000
ALPHA = -1.5
BETA = -0.5


def _final_pallas(pooled, lin_W, lin_b):
    def body(p_ref, w_ref, b_ref, o_ref):
        o_ref[...] = jnp.sum(p_ref[...] * w_ref[...].T, axis=1, keepdims=True)[0] + b_ref[...]

    return pl.pallas_call(
        body,
        out_shape=jax.ShapeDtypeStruct((1,), jnp.float32),
    )(pooled.reshape(1, -1), lin_W, lin_b)


def kernel(x_0, edge_index, W0_l0, b0_l0, W1_l0, b1_l0, W0_l1, b0_l1, W1_l1, b1_l1, lin_W, lin_b):
    src = edge_index[0]
    dst = edge_index[1]
    loops = jnp.arange(N, dtype=src.dtype)
    rows = jnp.concatenate([dst, loops])
    cols = jnp.concatenate([src, loops])
    ones = jnp.ones(rows.shape[0], dtype=jnp.float32)
    d_E = jax.ops.segment_sum(ones, cols, num_segments=EH)
    d_N = jax.ops.segment_sum(ones, rows, num_segments=N)
    edge_card = d_E ** ALPHA
    node_card = d_N ** BETA
    row_denom = jax.ops.segment_sum(edge_card[cols], rows, num_segments=N)
    col_denom = jax.ops.segment_sum(node_card[rows], cols, num_segments=EH)
    v10 = edge_card[cols] / row_denom[rows]
    v01 = node_card[rows] / col_denom[cols]

    def hnhn_layer(x, W0, b0, W1, b1):
        xm = x @ W0
        x1 = jax.ops.segment_sum(v01[:, None] * xm[rows], cols, num_segments=EH)
        x1 = jax.nn.relu(x1 + b0)
        xm2 = x1 @ W1
        x0 = jax.ops.segment_sum(v10[:, None] * xm2[cols], rows, num_segments=N)
        x0 = jax.nn.relu(x0 + b1)
        return x0, x1

    x, _ = hnhn_layer(x_0, W0_l0, b0_l0, W1_l0, b1_l0)
    x, _ = hnhn_layer(x, W0_l1, b0_l1, W1_l1, b1_l1)
    pooled = jnp.max(x, axis=0)
    return _final_pallas(pooled, lin_W, lin_b)


# speedup vs baseline: 16.0186x; 16.0160x over previous
import functools

import jax
import jax.numpy as jnp
from jax import lax
from jax.experimental import pallas as pl
from jax.experimental.pallas import tpu as pltpu
from jax.experimental.pallas import tpu_sc as plsc

N = 10000
E = 320000
D = 128

NC = 2
NS = 16
NW = NC * NS
EPW = E // NW
BLK = 100
NBLK = EPW // BLK

NSEG_PAD = 10240
RPS = NSEG_PAD // NS
NP = NSEG_PAD

f32 = jnp.float32
i32 = jnp.int32


def _mesh():
    return plsc.VectorSubcoreMesh(core_axis_name="c", subcore_axis_name="s",
                                  num_cores=NC, num_subcores=NS)


def _sc_degrees(src, dst):
    def body(src_hbm, dst_hbm, out_hbm, src_v, dst_v, hout, hin, spm, tmp_v, acc_v):
        c = lax.axis_index("c")
        s = lax.axis_index("s")
        wid = s * NC + c
        pltpu.sync_copy(src_hbm.at[pl.ds(wid * EPW, EPW)], src_v)
        pltpu.sync_copy(dst_hbm.at[pl.ds(wid * EPW, EPW)], dst_v)

        zeros16 = jnp.zeros((16,), f32)
        def zbody(i, _):
            hout[pl.ds(i * 16, 16)] = zeros16
            hin[pl.ds(i * 16, 16)] = zeros16
            return 0
        lax.fori_loop(0, NSEG_PAD // 16, zbody, 0)

        ones16 = jnp.ones((16,), f32)
        def sbody(i, _):
            sv = src_v[pl.ds(i * 16, 16)]
            plsc.addupdate_scatter(hout, [sv], ones16)
            dv = dst_v[pl.ds(i * 16, 16)]
            plsc.addupdate_scatter(hin, [dv], ones16)
            return 0
        lax.fori_loop(0, EPW // 16, sbody, 0)

        pltpu.sync_copy(hout, spm.at[0, s])
        pltpu.sync_copy(hin, spm.at[1, s])
        plsc.subcore_barrier()

        for d in range(2):
            pltpu.sync_copy(spm.at[d, 0, pl.ds(s * RPS, RPS)], acc_v)
            for k in range(1, NS):
                pltpu.sync_copy(spm.at[d, k, pl.ds(s * RPS, RPS)], tmp_v)
                def abody(j, _):
                    acc_v[pl.ds(j * 16, 16)] = (acc_v[pl.ds(j * 16, 16)]
                                                + tmp_v[pl.ds(j * 16, 16)])
                    return 0
                lax.fori_loop(0, RPS // 16, abody, 0)
            pltpu.sync_copy(acc_v,
                            out_hbm.at[pl.ds((c * 2 + d) * NSEG_PAD + s * RPS, RPS)])

    return pl.kernel(
        body,
        out_type=jax.ShapeDtypeStruct((NC * 2 * NSEG_PAD,), f32),
        mesh=_mesh(),
        compiler_params=pltpu.CompilerParams(needs_layout_passes=False),
        scratch_types=[
            pltpu.VMEM((EPW,), i32),
            pltpu.VMEM((EPW,), i32),
            pltpu.VMEM((NSEG_PAD,), f32),
            pltpu.VMEM((NSEG_PAD,), f32),
            pltpu.VMEM_SHARED((2, NS, NSEG_PAD), f32),
            pltpu.VMEM((RPS,), f32),
            pltpu.VMEM((RPS,), f32),
        ],
    )(src, dst)


def _sc_denoms(src, dst, ecard, ncard):
    def body(src_hbm, dst_hbm, ec_hbm, nc_hbm, out_hbm,
             src_v, dst_v, ec_v, nc_v, hrd, hcd, spm, tmp_v, acc_v):
        c = lax.axis_index("c")
        s = lax.axis_index("s")
        wid = s * NC + c
        pltpu.sync_copy(src_hbm.at[pl.ds(wid * EPW, EPW)], src_v)
        pltpu.sync_copy(dst_hbm.at[pl.ds(wid * EPW, EPW)], dst_v)
        pltpu.sync_copy(ec_hbm, ec_v)
        pltpu.sync_copy(nc_hbm, nc_v)

        zeros16 = jnp.zeros((16,), f32)
        def zbody(i, _):
            hrd[pl.ds(i * 16, 16)] = zeros16
            hcd[pl.ds(i * 16, 16)] = zeros16
            return 0
        lax.fori_loop(0, NSEG_PAD // 16, zbody, 0)

        def sbody(i, _):
            sv = src_v[pl.ds(i * 16, 16)]
            dv = dst_v[pl.ds(i * 16, 16)]
            ec = plsc.load_gather(ec_v, [sv])
            plsc.addupdate_scatter(hrd, [dv], ec)
            nc = plsc.load_gather(nc_v, [dv])
            plsc.addupdate_scatter(hcd, [sv], nc)
            return 0
        lax.fori_loop(0, EPW // 16, sbody, 0)

        pltpu.sync_copy(hrd, spm.at[0, s])
        pltpu.sync_copy(hcd, spm.at[1, s])
        plsc.subcore_barrier()

        for d in range(2):
            pltpu.sync_copy(spm.at[d, 0, pl.ds(s * RPS, RPS)], acc_v)
            for k in range(1, NS):
                pltpu.sync_copy(spm.at[d, k, pl.ds(s * RPS, RPS)], tmp_v)
                def abody(j, _):
                    acc_v[pl.ds(j * 16, 16)] = (acc_v[pl.ds(j * 16, 16)]
                                                + tmp_v[pl.ds(j * 16, 16)])
                    return 0
                lax.fori_loop(0, RPS // 16, abody, 0)
            pltpu.sync_copy(acc_v,
                            out_hbm.at[pl.ds((c * 2 + d) * NSEG_PAD + s * RPS, RPS)])

    return pl.kernel(
        body,
        out_type=jax.ShapeDtypeStruct((NC * 2 * NSEG_PAD,), f32),
        mesh=_mesh(),
        compiler_params=pltpu.CompilerParams(needs_layout_passes=False),
        scratch_types=[
            pltpu.VMEM((EPW,), i32),
            pltpu.VMEM((EPW,), i32),
            pltpu.VMEM((NP,), f32),
            pltpu.VMEM((NP,), f32),
            pltpu.VMEM((NSEG_PAD,), f32),
            pltpu.VMEM((NSEG_PAD,), f32),
            pltpu.VMEM_SHARED((2, NS, NSEG_PAD), f32),
            pltpu.VMEM((RPS,), f32),
            pltpu.VMEM((RPS,), f32),
        ],
    )(src, dst, ecard, ncard)


def _sc_spmm(xm, gidx, sidx, zeros2d):
    def body(xm_hbm, gidx_hbm, sidx_hbm, z_hbm, out_hbm,
             gidx_v, sidx_v, buf, acc, sem0, sem1):
        c = lax.axis_index("c")
        s = lax.axis_index("s")
        wid = s * NC + c
        pltpu.sync_copy(z_hbm.at[pl.ds(s * RPS, RPS)], acc.at[pl.ds(s * RPS, RPS)])
        plsc.subcore_barrier()

        HB = NBLK // 2
        for h in range(2):
            pltpu.sync_copy(gidx_hbm.at[wid, h], gidx_v)
            pltpu.sync_copy(sidx_hbm.at[wid, h], sidx_v)
            pltpu.async_copy(xm_hbm.at[gidx_v.at[0]], buf.at[0], sem0)

            def lbody(i, _):
                b0 = 2 * i
                b1 = 2 * i + 1
                pltpu.make_async_copy(xm_hbm.at[gidx_v.at[b0]], buf.at[0], sem0).wait()
                pltpu.async_copy(xm_hbm.at[gidx_v.at[b1]], buf.at[1], sem1)
                pltpu.sync_copy(buf.at[0], acc.at[sidx_v.at[b0]], add=True)
                pltpu.make_async_copy(xm_hbm.at[gidx_v.at[b1]], buf.at[1], sem1).wait()
                @pl.when(i < HB // 2 - 1)
                def _():
                    pltpu.async_copy(xm_hbm.at[gidx_v.at[b0 + 2]], buf.at[0], sem0)
                pltpu.sync_copy(buf.at[1], acc.at[sidx_v.at[b1]], add=True)
                return 0

            lax.fori_loop(0, HB // 2, lbody, 0)
        plsc.subcore_barrier()
        pltpu.sync_copy(acc.at[pl.ds(s * RPS, RPS)],
                        out_hbm.at[c, pl.ds(s * RPS, RPS)])

    return pl.kernel(
        body,
        out_type=jax.ShapeDtypeStruct((NC, NSEG_PAD, D), f32),
        mesh=_mesh(),
        compiler_params=pltpu.CompilerParams(needs_layout_passes=False),
        scratch_types=[
            pltpu.VMEM((NBLK // 2, BLK), i32),
            pltpu.VMEM((NBLK // 2, BLK), i32),
            pltpu.VMEM((2, BLK, D), f32),
            pltpu.VMEM_SHARED((NSEG_PAD, D), f32),
            pltpu.SemaphoreType.DMA,
            pltpu.SemaphoreType.DMA,
        ],
    )(xm, gidx, sidx, zeros2d)


RBLK = 1024
GRID = NP // RBLK


def _tc_cards_prescale(degp, x0, W0):
    def body(degp_ref, x_ref, w_ref, ncard_ref, ecard_ref, xmp_ref):
        i = pl.program_id(0)
        sl = pl.ds(i * RBLK, RBLK)
        outdeg = degp_ref[0, 0, sl] + degp_ref[1, 0, sl]
        indeg = degp_ref[0, 1, sl] + degp_ref[1, 1, sl]
        dE = outdeg + 1.0
        dN = indeg + 1.0
        ncard = lax.rsqrt(dN)
        ecard = lax.rsqrt(dE) / dE
        ncard_ref[sl] = ncard
        ecard_ref[sl] = ecard
        xm = jnp.dot(x_ref[...], w_ref[...], preferred_element_type=f32)
        xmp_ref[...] = xm * ncard[:, None]

    return pl.pallas_call(
        body,
        grid=(GRID,),
        in_specs=[
            pl.BlockSpec((NC, 2, NP), lambda i: (0, 0, 0)),
            pl.BlockSpec((RBLK, D), lambda i: (i, 0)),
            pl.BlockSpec((D, D), lambda i: (0, 0)),
        ],
        out_specs=[
            pl.BlockSpec((NP,), lambda i: (0,)),
            pl.BlockSpec((NP,), lambda i: (0,)),
            pl.BlockSpec((RBLK, D), lambda i: (i, 0)),
        ],
        out_shape=[
            jax.ShapeDtypeStruct((NP,), f32),
            jax.ShapeDtypeStruct((NP,), f32),
            jax.ShapeDtypeStruct((NP, D), f32),
        ],
    )(degp, x0, W0)


def _tc_mid(q, xmp, denp, dslot, selfcard, b, W, nextcard):
    def body(q_ref, xmp_ref, dp_ref, sc_ref, b_ref, w_ref, nc_ref, out_ref):
        i = pl.program_id(0)
        sl = pl.ds(i * RBLK, RBLK)
        den = dp_ref[0, dslot, sl] + dp_ref[1, dslot, sl] + sc_ref[sl]
        inv = 1.0 / den
        y = (q_ref[0] + q_ref[1] + xmp_ref[...]) * inv[:, None] + b_ref[...][None, :]
        y = jnp.maximum(y, 0.0)
        out_ref[...] = jnp.dot(y, w_ref[...], preferred_element_type=f32) * nc_ref[sl][:, None]

    return pl.pallas_call(
        body,
        grid=(GRID,),
        in_specs=[
            pl.BlockSpec((NC, RBLK, D), lambda i: (0, i, 0)),
            pl.BlockSpec((RBLK, D), lambda i: (i, 0)),
            pl.BlockSpec((NC, 2, NP), lambda i: (0, 0, 0)),
            pl.BlockSpec((NP,), lambda i: (0,)),
            pl.BlockSpec((D,), lambda i: (0,)),
            pl.BlockSpec((D, D), lambda i: (0, 0)),
            pl.BlockSpec((NP,), lambda i: (0,)),
        ],
        out_specs=pl.BlockSpec((RBLK, D), lambda i: (i, 0)),
        out_shape=jax.ShapeDtypeStruct((NP, D), f32),
    )(q, xmp, denp, selfcard, b, W, nextcard)


def _tc_final(q, xmp, denp, dslot, selfcard, b, linW, linb):
    def body(q_ref, xmp_ref, dp_ref, sc_ref, b_ref, lw_ref, lb_ref, out_ref, macc):
        i = pl.program_id(0)
        sl = pl.ds(i * RBLK, RBLK)
        den = dp_ref[0, dslot, sl] + dp_ref[1, dslot, sl] + sc_ref[sl]
        inv = 1.0 / den
        x = (q_ref[0] + q_ref[1] + xmp_ref[...]) * inv[:, None] + b_ref[...][None, :]
        x = jnp.maximum(x, 0.0)
        rowid = i * RBLK + lax.broadcasted_iota(i32, (RBLK, D), 0)
        x = jnp.where(rowid < N, x, 0.0)
        m = jnp.max(x, axis=0, keepdims=True)

        @pl.when(i == 0)
        def _():
            macc[...] = m

        @pl.when(i > 0)
        def _():
            macc[...] = jnp.maximum(macc[...], m)

        @pl.when(i == pl.num_programs(0) - 1)
        def _():
            out_ref[...] = (jnp.sum(macc[...] * lw_ref[...].T, axis=1, keepdims=True)
                            + lb_ref[...])

    return pl.pallas_call(
        body,
        grid=(GRID,),
        in_specs=[
            pl.BlockSpec((NC, RBLK, D), lambda i: (0, i, 0)),
            pl.BlockSpec((RBLK, D), lambda i: (i, 0)),
            pl.BlockSpec((NC, 2, NP), lambda i: (0, 0, 0)),
            pl.BlockSpec((NP,), lambda i: (0,)),
            pl.BlockSpec((D,), lambda i: (0,)),
            pl.BlockSpec((D, 1), lambda i: (0, 0)),
            pl.BlockSpec((1, 1), lambda i: (0, 0)),
        ],
        out_specs=pl.BlockSpec((1, 1), lambda i: (0, 0)),
        out_shape=jax.ShapeDtypeStruct((1, 1), f32),
        scratch_shapes=[pltpu.VMEM((1, D), f32)],
    )(q, xmp, denp, selfcard, b, linW, linb)


def kernel(x_0, edge_index, W0_l0, b0_l0, W1_l0, b1_l0,
           W0_l1, b0_l1, W1_l1, b1_l1, lin_W, lin_b):
    src = edge_index[0].astype(i32)
    dst = edge_index[1].astype(i32)
    src3 = src.reshape(NW, 2, NBLK // 2, BLK)
    dst3 = dst.reshape(NW, 2, NBLK // 2, BLK)
    zeros2d = jnp.zeros((NSEG_PAD, D), f32)
    x0p = jnp.concatenate([x_0, jnp.zeros((NP - N, D), f32)], axis=0)

    degp = _sc_degrees(src, dst).reshape(NC, 2, NSEG_PAD)
    ncard, ecard, xm0p = _tc_cards_prescale(degp, x0p, W0_l0)
    denp = _sc_denoms(src, dst, ecard, ncard).reshape(NC, 2, NSEG_PAD)

    q1 = _sc_spmm(xm0p, dst3, src3, zeros2d)
    xm1p = _tc_mid(q1, xm0p, denp, 1, ncard, b0_l0, W1_l0, ecard)
    q0 = _sc_spmm(xm1p, src3, dst3, zeros2d)
    xm2p = _tc_mid(q0, xm1p, denp, 0, ecard, b1_l0, W0_l1, ncard)
    q1b = _sc_spmm(xm2p, dst3, src3, zeros2d)
    xm3p = _tc_mid(q1b, xm2p, denp, 1, ncard, b0_l1, W1_l1, ecard)
    q0b = _sc_spmm(xm3p, src3, dst3, zeros2d)
    out = _tc_final(q0b, xm3p, denp, 0, ecard, b1_l1, lin_W, lin_b.reshape(1, 1))
    return out.reshape(1)
